# Initial kernel scaffold; baseline (speedup 1.0000x reference)
import functools

import jax
import jax.numpy as jnp
from jax import lax
from jax.experimental import pallas as pl
from jax.experimental.pallas import tpu as pltpu
from jax.experimental.pallas import tpu_sc as plsc

BATCH = 16384
HIST = 50
EMBED_DIM = 32
B_TOTAL = BATCH * HIST

NUM_CORES = 2
NUM_SUBCORES = 16
NW = NUM_CORES * NUM_SUBCORES
B_PER_W = B_TOTAL // NW
CHUNK = 1024
N_CHUNKS = B_PER_W // CHUNK

_mesh = plsc.VectorSubcoreMesh(core_axis_name="c", subcore_axis_name="s")


@functools.partial(
    pl.kernel,
    mesh=_mesh,
    out_type=jax.ShapeDtypeStruct((B_TOTAL, EMBED_DIM), jnp.float32),
    scratch_types=[
        pltpu.VMEM((CHUNK,), jnp.int32),
        pltpu.VMEM((CHUNK, EMBED_DIM), jnp.float32),
        pltpu.SemaphoreType.DMA,
    ],
)
def _gather_kernel(idx_hbm, table_hbm, out_hbm, idx_v, rows_v, sem):
    wid = lax.axis_index("s") * NUM_CORES + lax.axis_index("c")
    base = wid * B_PER_W

    def body(i, carry):
        off = base + i * CHUNK
        pltpu.sync_copy(idx_hbm.at[pl.ds(off, CHUNK)], idx_v)
        pltpu.async_copy(table_hbm.at[idx_v], rows_v, sem).wait()
        pltpu.sync_copy(rows_v, out_hbm.at[pl.ds(off, CHUNK)])
        return carry

    lax.fori_loop(0, N_CHUNKS, body, 0)


def kernel(inputs, table):
    flat = inputs.reshape(-1)
    out = _gather_kernel(flat, table)
    return out.reshape(BATCH, HIST, EMBED_DIM)


# speedup vs baseline: 1.0936x; 1.0936x over previous
import functools

import jax
import jax.numpy as jnp
from jax import lax
from jax.experimental import pallas as pl
from jax.experimental.pallas import tpu as pltpu
from jax.experimental.pallas import tpu_sc as plsc

BATCH = 16384
HIST = 50
EMBED_DIM = 32
B_TOTAL = BATCH * HIST

NUM_CORES = 2
NUM_SUBCORES = 16
NW = NUM_CORES * NUM_SUBCORES
B_PER_W = B_TOTAL // NW
CHUNK = 1024
N_CHUNKS = B_PER_W // CHUNK

_mesh = plsc.VectorSubcoreMesh(core_axis_name="c", subcore_axis_name="s")


@functools.partial(
    pl.kernel,
    mesh=_mesh,
    out_type=jax.ShapeDtypeStruct((B_TOTAL, EMBED_DIM), jnp.float32),
    scratch_types=[
        pltpu.VMEM((CHUNK,), jnp.int32),
        pltpu.VMEM((CHUNK, EMBED_DIM), jnp.float32),
        pltpu.SemaphoreType.DMA,
    ],
    compiler_params=pltpu.CompilerParams(use_tc_tiling_on_sc=False),
)
def _gather_kernel(idx_hbm, table_hbm, out_hbm, idx_v, rows_v, sem):
    wid = lax.axis_index("s") * NUM_CORES + lax.axis_index("c")
    base = wid * B_PER_W

    def body(i, carry):
        off = base + i * CHUNK
        pltpu.sync_copy(idx_hbm.at[pl.ds(off, CHUNK)], idx_v)
        pltpu.async_copy(table_hbm.at[idx_v], rows_v, sem).wait()
        pltpu.sync_copy(rows_v, out_hbm.at[pl.ds(off, CHUNK)])
        return carry

    lax.fori_loop(0, N_CHUNKS, body, 0)


def kernel(inputs, table):
    flat = inputs.reshape(-1)
    out = _gather_kernel(flat, table)
    return out.reshape(BATCH, HIST, EMBED_DIM)


# speedup vs baseline: 1.1126x; 1.0174x over previous
import functools

import jax
import jax.numpy as jnp
from jax import lax
from jax.experimental import pallas as pl
from jax.experimental.pallas import tpu as pltpu
from jax.experimental.pallas import tpu_sc as plsc

BATCH = 16384
HIST = 50
EMBED_DIM = 32
B_TOTAL = BATCH * HIST

NUM_CORES = 2
NUM_SUBCORES = 16
NW = NUM_CORES * NUM_SUBCORES
B_PER_W = B_TOTAL // NW

CHUNK = 320
K = 8
D = 4
N_CHUNKS = B_PER_W // CHUNK
G_STEADY = N_CHUNKS // K

_mesh = plsc.VectorSubcoreMesh(core_axis_name="c", subcore_axis_name="s")


@functools.partial(
    pl.kernel,
    mesh=_mesh,
    out_type=jax.ShapeDtypeStruct((B_TOTAL, EMBED_DIM), jnp.float32),
    scratch_types=[
        pltpu.VMEM((B_PER_W,), jnp.int32),
    ]
    + [pltpu.VMEM((CHUNK, EMBED_DIM), jnp.float32) for _ in range(K)]
    + [
        pltpu.SemaphoreType.DMA((K,)),
        pltpu.SemaphoreType.DMA((K,)),
    ],
    compiler_params=pltpu.CompilerParams(use_tc_tiling_on_sc=False),
)
def _gather_kernel(idx_hbm, table_hbm, out_hbm, idx_all,
                   r0, r1, r2, r3, r4, r5, r6, r7, gs, ws):
    rows = [r0, r1, r2, r3, r4, r5, r6, r7]
    wid = lax.axis_index("s") * NUM_CORES + lax.axis_index("c")
    base = wid * B_PER_W

    pltpu.sync_copy(idx_hbm.at[pl.ds(base, B_PER_W)], idx_all)

    def start_gather(i, b):
        pltpu.make_async_copy(
            table_hbm.at[idx_all.at[pl.ds(i * CHUNK, CHUNK)]],
            rows[b], gs.at[b]).start()

    def wait_gather(b):
        pltpu.make_async_copy(
            table_hbm.at[idx_all.at[pl.ds(0, CHUNK)]],
            rows[b], gs.at[b]).wait()

    def start_write(j, b):
        pltpu.make_async_copy(
            rows[b], out_hbm.at[pl.ds(base + j * CHUNK, CHUNK)],
            ws.at[b]).start()

    def wait_write(b):
        pltpu.make_async_copy(
            rows[b], out_hbm.at[pl.ds(base, CHUNK)], ws.at[b]).wait()

    for i in range(K):
        start_gather(i, i)
        if i >= D:
            j = i - D
            wait_gather(j % K)
            start_write(j, j % K)

    def group(g, carry):
        for b in range(K):
            i = g * K + b
            wait_write(b)
            start_gather(i, b)
            j = i - D
            bj = (b - D) % K
            wait_gather(bj)
            start_write(j, bj)
        return carry

    lax.fori_loop(1, G_STEADY, group, 0)

    for t in range(D):
        j = N_CHUNKS - D + t
        bj = j % K
        wait_gather(bj)
        start_write(j, bj)
    for b in range(K):
        wait_write(b)


def kernel(inputs, table):
    flat = inputs.reshape(-1)
    out = _gather_kernel(flat, table)
    return out.reshape(BATCH, HIST, EMBED_DIM)


# speedup vs baseline: 1.8145x; 1.6309x over previous
import functools

import jax
import jax.numpy as jnp
from jax import lax
from jax.experimental import pallas as pl
from jax.experimental.pallas import tpu as pltpu
from jax.experimental.pallas import tpu_sc as plsc

BATCH = 16384
HIST = 50
EMBED_DIM = 32
B_TOTAL = BATCH * HIST

NUM_CORES = 2
NUM_SUBCORES = 16
NW = NUM_CORES * NUM_SUBCORES
BW = BATCH // NW
SLAB = BW * HIST

_mesh = plsc.VectorSubcoreMesh(core_axis_name="c", subcore_axis_name="s")


@functools.partial(
    pl.kernel,
    mesh=_mesh,
    out_type=jax.ShapeDtypeStruct((HIST, 4, 1024, 128), jnp.float32),
    scratch_types=[
        pltpu.VMEM((SLAB,), jnp.int32),
        pltpu.VMEM((BW,), jnp.int32),
        pltpu.VMEM((BW,), jnp.int32),
        pltpu.VMEM((BW, EMBED_DIM), jnp.float32),
        pltpu.VMEM((BW, EMBED_DIM), jnp.float32),
        pltpu.VMEM((4, 32, 128), jnp.float32),
        pltpu.VMEM((4, 32, 128), jnp.float32),
        pltpu.SemaphoreType.DMA((2,)),
        pltpu.SemaphoreType.DMA((2,)),
    ],
    compiler_params=pltpu.CompilerParams(
        use_tc_tiling_on_sc=False, needs_layout_passes=False),
)
def _lookup_kernel(idx_hbm, table_hbm, out_hbm, slab,
                   iu0, iu1, g0, g1, t0, t1, gs, ws):
    iu = [iu0, iu1]
    g = [g0, g1]
    t = [t0, t1]
    wid = lax.axis_index("s") * NUM_CORES + lax.axis_index("c")

    pltpu.sync_copy(idx_hbm.at[pl.ds(wid * SLAB, SLAB)], slab)

    lane = lax.iota(jnp.int32, 16)
    lane50 = lane * 50
    tr_c = [(p * 16 + lane) >> 3 for p in range(2)]
    js_c = [(p * 16 + lane) & 7 for p in range(2)]

    def compact(h, p):
        def body(c, carry):
            vec = plsc.load_gather(slab, [lane50 + (c * 800 + h)])
            iu[p][pl.ds(c * 16, 16)] = vec
            return carry
        lax.fori_loop(0, 32, body, 0, unroll=4)

    def gather_start(p):
        pltpu.make_async_copy(table_hbm.at[iu[p]], g[p], gs.at[p]).start()

    def gather_wait(p):
        pltpu.make_async_copy(table_hbm.at[iu[p]], g[p], gs.at[p]).wait()

    def transpose(p):
        def body(r, carry):
            qbase = (r >> 7) * 8
            bl = jnp.full((16,), r & 127, jnp.int32)
            for par in range(2):
                vals = g[p][r, pl.ds(par * 16, 16)]
                plsc.store_scatter(t[p], [tr_c[par], js_c[par] + qbase, bl],
                                   vals)
            return carry
        lax.fori_loop(0, BW, body, 0, unroll=4)

    def out_start(h, p):
        pltpu.make_async_copy(
            t[p], out_hbm.at[h, :, pl.ds(32 * wid, 32), :], ws.at[p]).start()

    def out_wait(p):
        pltpu.make_async_copy(
            t[p], out_hbm.at[0, :, pl.ds(32 * wid, 32), :], ws.at[p]).wait()

    compact(0, 0)
    gather_start(0)
    for h in range(HIST):
        p = h % 2
        if h + 1 < HIST:
            compact(h + 1, 1 - p)
            gather_start(1 - p)
        gather_wait(p)
        if h >= 2:
            out_wait(p)
        transpose(p)
        out_start(h, p)
    out_wait(0 if HIST % 2 == 0 else 1)
    out_wait(1 if HIST % 2 == 0 else 0)


def kernel(inputs, table):
    flat = inputs.reshape(-1)
    out5 = _lookup_kernel(flat, table)
    return (out5.reshape(HIST, 4, 128, 8, 128)
                .transpose(2, 4, 0, 1, 3)
                .reshape(BATCH, HIST, EMBED_DIM))
